# baseline (speedup 1.0000x reference)
import functools

import jax
import jax.numpy as jnp
from jax import lax
from jax.experimental import pallas as pl
from jax.experimental.pallas import tpu as pltpu
from jax.experimental.pallas import tpu_sc as plsc

NC = 2
NS = 16
NW = NC * NS
CHUNK = 128
CHUNK2 = 64
DW = 16


def _sc_mesh():
    return plsc.VectorSubcoreMesh(core_axis_name="c", subcore_axis_name="s")


def _make_deg_kernel(npad, epad):
    cpw = epad // (NW * CHUNK2)
    rpt = npad // NS
    lag = 8

    @functools.partial(
        pl.kernel,
        out_type=jax.ShapeDtypeStruct((NC * npad, DW), jnp.float32),
        mesh=_sc_mesh(),
        scratch_types=[
            pltpu.VMEM((cpw, CHUNK2), jnp.int32),
            pltpu.VMEM((CHUNK2, DW), jnp.float32),
            pltpu.VMEM_SHARED((npad, DW), jnp.float32),
            pltpu.SemaphoreType.DMA,
        ],
    )
    def deg_kernel(aux_hbm, dst_hbm, out_hbm, dstbuf, rowbuf, accum, ssem):
        c = lax.axis_index("c")
        s = lax.axis_index("s")
        w = c * NS + s

        pltpu.sync_copy(aux_hbm.at[pl.ds(0, rpt)],
                        accum.at[pl.ds(s * rpt, rpt)])
        pltpu.sync_copy(dst_hbm.at[pl.ds(w * cpw, cpw)], dstbuf)
        pltpu.sync_copy(aux_hbm.at[pl.ds(rpt, CHUNK2)], rowbuf)
        plsc.subcore_barrier()

        for j in range(lag):
            pltpu.async_copy(rowbuf, accum.at[dstbuf.at[j]], ssem, add=True)

        @pl.loop(0, cpw - lag)
        def _(j):
            pltpu.make_async_copy(rowbuf, accum.at[dstbuf.at[j]], ssem).wait()
            pltpu.async_copy(rowbuf, accum.at[dstbuf.at[j + lag]], ssem,
                             add=True)

        for j in range(lag):
            pltpu.make_async_copy(rowbuf, accum.at[dstbuf.at[j]], ssem).wait()

        plsc.subcore_barrier()
        pltpu.sync_copy(accum.at[pl.ds(s * rpt, rpt)],
                        out_hbm.at[pl.ds(c * npad + s * rpt, rpt)])

    return deg_kernel


def _make_spass_kernel(npad, epad, dh):
    rpt = npad // NS
    ch = CHUNK2
    nbuf = 5
    sr = 4
    cpt2 = epad // (NS * ch)
    cpr = cpt2 // sr
    mid = cpr - nbuf
    assert mid % nbuf == 0 and cpr > 2 * nbuf

    @functools.partial(
        pl.kernel,
        out_type=jax.ShapeDtypeStruct((NC * npad, dh), jnp.float32),
        mesh=_sc_mesh(),
        scratch_types=[
            pltpu.VMEM((cpr * ch,), jnp.int32),
            pltpu.VMEM((cpr, ch), jnp.int32),
            pltpu.VMEM((nbuf, ch, dh), jnp.float32),
            pltpu.VMEM_SHARED((npad, dh), jnp.float32),
            [pltpu.SemaphoreType.DMA] * nbuf,
            [pltpu.SemaphoreType.DMA] * nbuf,
        ],
    )
    def spass_kernel(y_hbm, src2_hbm, dst_hbm, out_hbm, srcbuf, dstbuf,
                     rows, accum, gsems, ssems):
        c = lax.axis_index("c")
        s = lax.axis_index("s")

        def gather(j, b):
            pltpu.async_copy(
                y_hbm.at[srcbuf.at[pl.ds(j * ch, ch)]], rows.at[b], gsems[b])

        def gather_wait(b):
            pltpu.make_async_copy(
                y_hbm.at[srcbuf.at[pl.ds(0, ch)]], rows.at[b], gsems[b]).wait()

        def scatter(j, b):
            pltpu.async_copy(rows.at[b], accum.at[dstbuf.at[j]], ssems[b],
                             add=True)

        def scatter_wait(j, b):
            pltpu.make_async_copy(rows.at[b], accum.at[dstbuf.at[j]],
                                  ssems[b]).wait()

        pltpu.sync_copy(y_hbm.at[pl.ds(c * npad + s * rpt, rpt)],
                        accum.at[pl.ds(s * rpt, rpt)])
        plsc.subcore_barrier()

        for r in range(sr):
            pltpu.sync_copy(
                src2_hbm.at[pl.ds(
                    c * epad + (s * cpt2 + r * cpr) * ch, cpr * ch)],
                srcbuf)
            pltpu.sync_copy(dst_hbm.at[pl.ds(s * cpt2 + r * cpr, cpr)], dstbuf)

            for j in range(nbuf - 1):
                gather(j, j)

            def step(j, b, do_swait, do_gather):
                gather_wait(b)
                scatter(j, b)
                b2 = (b + nbuf - 1) % nbuf
                if do_swait:
                    scatter_wait(j - 1, b2)
                if do_gather:
                    gather(j + nbuf - 1, b2)

            step(0, 0, False, True)

            @pl.loop(0, mid // nbuf)
            def _(it):
                for u in range(nbuf):
                    jj = 1 + it * nbuf + u
                    step(jj, (1 + u) % nbuf, True, True)

            for j in range(cpr - nbuf + 1, cpr):
                step(j, j % nbuf, True, False)
            scatter_wait(cpr - 1, (cpr - 1) % nbuf)

        plsc.subcore_barrier()
        pltpu.sync_copy(accum.at[pl.ds(s * rpt, rpt)],
                        out_hbm.at[pl.ds(c * npad + s * rpt, rpt)])

    return spass_kernel


def _dinv_block(dg_ref):
    d0 = dg_ref[0]
    d1 = dg_ref[1]
    deg = d0[:, 0:1] + d1[:, 0:1] + 1.0
    return lax.rsqrt(jnp.maximum(deg, 1.0))


def _make_tc_y1(npad, d, br):
    def body(x_ref, w_ref, dg_ref, y_ref):
        dinv = _dinv_block(dg_ref)
        xw = jnp.dot(x_ref[...], w_ref[...], preferred_element_type=jnp.float32)
        y = xw * dinv
        y_ref[0] = y[:, : d // 2]
        y_ref[1] = y[:, d // 2:]

    return pl.pallas_call(
        body,
        grid=(npad // br,),
        in_specs=[
            pl.BlockSpec((br, d), lambda i: (i, 0)),
            pl.BlockSpec((d, d), lambda i: (0, 0)),
            pl.BlockSpec((NC, br, DW), lambda i: (0, i, 0)),
        ],
        out_specs=pl.BlockSpec((NC, br, d // 2), lambda i: (0, i, 0)),
        out_shape=jax.ShapeDtypeStruct((NC, npad, d // 2), jnp.float32),
    )


def _make_tc_mid(npad, d, br):
    def body(s_ref, dg_ref, b_ref, w_ref, y_ref):
        dinv = _dinv_block(dg_ref)
        sfull = jnp.concatenate([s_ref[0], s_ref[1]], axis=1)
        h = jnp.maximum(sfull * dinv + b_ref[...], 0.0)
        y = jnp.dot(h, w_ref[...], preferred_element_type=jnp.float32) * dinv
        y_ref[0] = y[:, : d // 2]
        y_ref[1] = y[:, d // 2:]

    return pl.pallas_call(
        body,
        grid=(npad // br,),
        in_specs=[
            pl.BlockSpec((NC, br, d // 2), lambda i: (0, i, 0)),
            pl.BlockSpec((NC, br, DW), lambda i: (0, i, 0)),
            pl.BlockSpec((1, d), lambda i: (0, 0)),
            pl.BlockSpec((d, d), lambda i: (0, 0)),
        ],
        out_specs=pl.BlockSpec((NC, br, d // 2), lambda i: (0, i, 0)),
        out_shape=jax.ShapeDtypeStruct((NC, npad, d // 2), jnp.float32),
    )


def _make_tc_final(npad, d, br):
    def body(s_ref, dg_ref, b_ref, h_ref):
        dinv = _dinv_block(dg_ref)
        sfull = jnp.concatenate([s_ref[0], s_ref[1]], axis=1)
        h_ref[...] = jnp.maximum(sfull * dinv + b_ref[...], 0.0)

    return pl.pallas_call(
        body,
        grid=(npad // br,),
        in_specs=[
            pl.BlockSpec((NC, br, d // 2), lambda i: (0, i, 0)),
            pl.BlockSpec((NC, br, DW), lambda i: (0, i, 0)),
            pl.BlockSpec((1, d), lambda i: (0, 0)),
        ],
        out_specs=pl.BlockSpec((br, d), lambda i: (i, 0)),
        out_shape=jax.ShapeDtypeStruct((npad, d), jnp.float32),
    )


def kernel(x, edge_index, W1, b1, W2, b2):
    n, d = x.shape
    dh = d // 2
    e = edge_index.shape[1]
    npad = -(-(n + 1) // (NS * CHUNK)) * (NS * CHUNK)
    epad = -(-e // (NW * CHUNK)) * (NW * CHUNK)
    br = 512
    assert npad % br == 0

    src = edge_index[0].astype(jnp.int32)
    dst = edge_index[1].astype(jnp.int32)
    padlen = epad - e
    src = jnp.concatenate([src, jnp.full((padlen,), n, jnp.int32)])
    dst = jnp.concatenate([dst, jnp.full((padlen,), n, jnp.int32)])
    src2 = jnp.concatenate([src, src + npad])
    dst2d = dst.reshape(epad // CHUNK2, CHUNK2)

    xpad = jnp.zeros((npad, d), x.dtype).at[:n].set(x)
    aux = jnp.concatenate([jnp.zeros((npad // NS, DW), jnp.float32),
                           jnp.ones((CHUNK2, DW), jnp.float32)])

    deg_k = _make_deg_kernel(npad, epad)
    spass_k = _make_spass_kernel(npad, epad, dh)
    tc_y1 = _make_tc_y1(npad, d, br)
    tc_mid = _make_tc_mid(npad, d, br)
    tc_final = _make_tc_final(npad, d, br)

    dg = deg_k(aux, dst2d).reshape(NC, npad, DW)
    b1r = b1.reshape(1, d)
    b2r = b2.reshape(1, d)

    y1 = tc_y1(xpad, W1, dg)
    s1 = spass_k(y1.reshape(NC * npad, dh), src2, dst2d)
    y2 = tc_mid(s1.reshape(NC, npad, dh), dg, b1r, W2)
    s2 = spass_k(y2.reshape(NC * npad, dh), src2, dst2d)
    h = tc_final(s2.reshape(NC, npad, dh), dg, b2r)
    return h[:n]


# speedup vs baseline: 1.2322x; 1.2322x over previous
import functools

import jax
import jax.numpy as jnp
from jax import lax
from jax.experimental import pallas as pl
from jax.experimental.pallas import tpu as pltpu
from jax.experimental.pallas import tpu_sc as plsc

NC = 2
NS = 16
NW = NC * NS
CHUNK = 128
CHUNK2 = 64
DW = 16


def _sc_mesh():
    return plsc.VectorSubcoreMesh(core_axis_name="c", subcore_axis_name="s")


def _make_deg_kernel(npad, epad):
    cpw = epad // (NW * CHUNK2)
    rpt = npad // NS
    nzc = rpt // CHUNK
    lag = 8

    @functools.partial(
        pl.kernel,
        out_type=jax.ShapeDtypeStruct((NC * npad, DW), jnp.float32),
        mesh=_sc_mesh(),
        scratch_types=[
            pltpu.VMEM((cpw, CHUNK2), jnp.int32),
            pltpu.VMEM((CHUNK2, DW), jnp.float32),
            pltpu.VMEM_SHARED((npad, DW), jnp.float32),
            pltpu.SemaphoreType.DMA,
        ],
    )
    def deg_kernel(aux_hbm, dst_hbm, out_hbm, dstbuf, rowbuf, accum, ssem):
        c = lax.axis_index("c")
        s = lax.axis_index("s")
        w = c * NS + s

        @pl.loop(0, nzc)
        def _(k):
            pltpu.sync_copy(aux_hbm.at[pl.ds(0, CHUNK)],
                            accum.at[pl.ds(s * rpt + k * CHUNK, CHUNK)])

        pltpu.sync_copy(dst_hbm.at[pl.ds(w * cpw, cpw)], dstbuf)
        pltpu.sync_copy(aux_hbm.at[pl.ds(CHUNK, CHUNK2)], rowbuf)
        plsc.subcore_barrier()

        for j in range(lag):
            pltpu.async_copy(rowbuf, accum.at[dstbuf.at[j]], ssem, add=True)

        @pl.loop(0, cpw - lag)
        def _(j):
            pltpu.make_async_copy(rowbuf, accum.at[dstbuf.at[j]], ssem).wait()
            pltpu.async_copy(rowbuf, accum.at[dstbuf.at[j + lag]], ssem,
                             add=True)

        for j in range(lag):
            pltpu.make_async_copy(rowbuf, accum.at[dstbuf.at[j]], ssem).wait()

        plsc.subcore_barrier()
        pltpu.sync_copy(accum.at[pl.ds(s * rpt, rpt)],
                        out_hbm.at[pl.ds(c * npad + s * rpt, rpt)])

    return deg_kernel


def _make_spass_kernel(npad, epad, dh):
    rpt = npad // NS
    ch = CHUNK2
    nbuf = 5
    sr = 4
    cpt2 = epad // (NS * ch)
    cpr = cpt2 // sr
    mid = cpr - nbuf
    assert mid % nbuf == 0 and cpr > 2 * nbuf

    @functools.partial(
        pl.kernel,
        out_type=jax.ShapeDtypeStruct((NC * npad, dh), jnp.float32),
        mesh=_sc_mesh(),
        scratch_types=[
            pltpu.VMEM((cpr * ch,), jnp.int32),
            pltpu.VMEM((cpr, ch), jnp.int32),
            pltpu.VMEM((nbuf, ch, dh), jnp.float32),
            pltpu.VMEM_SHARED((npad, dh), jnp.float32),
            [pltpu.SemaphoreType.DMA] * nbuf,
            [pltpu.SemaphoreType.DMA] * nbuf,
        ],
    )
    def spass_kernel(y_hbm, src2_hbm, dst_hbm, out_hbm, srcbuf, dstbuf,
                     rows, accum, gsems, ssems):
        c = lax.axis_index("c")
        s = lax.axis_index("s")

        def gather(j, b):
            pltpu.async_copy(
                y_hbm.at[srcbuf.at[pl.ds(j * ch, ch)]], rows.at[b], gsems[b])

        def gather_wait(b):
            pltpu.make_async_copy(
                y_hbm.at[srcbuf.at[pl.ds(0, ch)]], rows.at[b], gsems[b]).wait()

        def scatter(j, b):
            pltpu.async_copy(rows.at[b], accum.at[dstbuf.at[j]], ssems[b],
                             add=True)

        def scatter_wait(j, b):
            pltpu.make_async_copy(rows.at[b], accum.at[dstbuf.at[j]],
                                  ssems[b]).wait()

        pltpu.sync_copy(y_hbm.at[pl.ds(c * npad + s * rpt, rpt)],
                        accum.at[pl.ds(s * rpt, rpt)])
        plsc.subcore_barrier()

        for r in range(sr):
            pltpu.sync_copy(
                src2_hbm.at[pl.ds(
                    c * epad + (s * cpt2 + r * cpr) * ch, cpr * ch)],
                srcbuf)
            pltpu.sync_copy(dst_hbm.at[pl.ds(s * cpt2 + r * cpr, cpr)], dstbuf)

            for j in range(nbuf - 1):
                gather(j, j)

            def step(j, b, do_swait, do_gather):
                gather_wait(b)
                scatter(j, b)
                b2 = (b + nbuf - 1) % nbuf
                if do_swait:
                    scatter_wait(j - 1, b2)
                if do_gather:
                    gather(j + nbuf - 1, b2)

            step(0, 0, False, True)

            @pl.loop(0, mid // nbuf)
            def _(it):
                for u in range(nbuf):
                    jj = 1 + it * nbuf + u
                    step(jj, (1 + u) % nbuf, True, True)

            for j in range(cpr - nbuf + 1, cpr):
                step(j, j % nbuf, True, False)
            scatter_wait(cpr - 1, (cpr - 1) % nbuf)

        plsc.subcore_barrier()
        pltpu.sync_copy(accum.at[pl.ds(s * rpt, rpt)],
                        out_hbm.at[pl.ds(c * npad + s * rpt, rpt)])

    return spass_kernel


def _dinv_block(dg_ref):
    d0 = dg_ref[0]
    d1 = dg_ref[1]
    deg = d0[:, 0:1] + d1[:, 0:1] + 1.0
    return lax.rsqrt(jnp.maximum(deg, 1.0))


def _make_tc_y1(npad, d, br):
    def body(x_ref, w_ref, dg_ref, y_ref):
        dinv = _dinv_block(dg_ref)
        xw = jnp.dot(x_ref[...], w_ref[...], preferred_element_type=jnp.float32)
        y = xw * dinv
        y_ref[0] = y[:, : d // 2]
        y_ref[1] = y[:, d // 2:]

    return pl.pallas_call(
        body,
        grid=(npad // br,),
        in_specs=[
            pl.BlockSpec((br, d), lambda i: (i, 0)),
            pl.BlockSpec((d, d), lambda i: (0, 0)),
            pl.BlockSpec((NC, br, DW), lambda i: (0, i, 0)),
        ],
        out_specs=pl.BlockSpec((NC, br, d // 2), lambda i: (0, i, 0)),
        out_shape=jax.ShapeDtypeStruct((NC, npad, d // 2), jnp.float32),
    )


def _make_tc_mid(npad, d, br):
    def body(s_ref, dg_ref, b_ref, w_ref, y_ref):
        dinv = _dinv_block(dg_ref)
        sfull = jnp.concatenate([s_ref[0], s_ref[1]], axis=1)
        h = jnp.maximum(sfull * dinv + b_ref[...], 0.0)
        y = jnp.dot(h, w_ref[...], preferred_element_type=jnp.float32) * dinv
        y_ref[0] = y[:, : d // 2]
        y_ref[1] = y[:, d // 2:]

    return pl.pallas_call(
        body,
        grid=(npad // br,),
        in_specs=[
            pl.BlockSpec((NC, br, d // 2), lambda i: (0, i, 0)),
            pl.BlockSpec((NC, br, DW), lambda i: (0, i, 0)),
            pl.BlockSpec((1, d), lambda i: (0, 0)),
            pl.BlockSpec((d, d), lambda i: (0, 0)),
        ],
        out_specs=pl.BlockSpec((NC, br, d // 2), lambda i: (0, i, 0)),
        out_shape=jax.ShapeDtypeStruct((NC, npad, d // 2), jnp.float32),
    )


def _make_tc_final(npad, d, br):
    def body(s_ref, dg_ref, b_ref, h_ref):
        dinv = _dinv_block(dg_ref)
        sfull = jnp.concatenate([s_ref[0], s_ref[1]], axis=1)
        h_ref[...] = jnp.maximum(sfull * dinv + b_ref[...], 0.0)

    return pl.pallas_call(
        body,
        grid=(npad // br,),
        in_specs=[
            pl.BlockSpec((NC, br, d // 2), lambda i: (0, i, 0)),
            pl.BlockSpec((NC, br, DW), lambda i: (0, i, 0)),
            pl.BlockSpec((1, d), lambda i: (0, 0)),
        ],
        out_specs=pl.BlockSpec((br, d), lambda i: (i, 0)),
        out_shape=jax.ShapeDtypeStruct((npad, d), jnp.float32),
    )


def kernel(x, edge_index, W1, b1, W2, b2):
    n, d = x.shape
    dh = d // 2
    e = edge_index.shape[1]
    npad = -(-(n + 1) // (NS * CHUNK)) * (NS * CHUNK)
    epad = -(-e // (NW * CHUNK)) * (NW * CHUNK)
    br = 512
    assert npad % br == 0

    src = edge_index[0].astype(jnp.int32)
    dst = edge_index[1].astype(jnp.int32)
    padlen = epad - e
    src = jnp.concatenate([src, jnp.full((padlen,), n, jnp.int32)])
    dst = jnp.concatenate([dst, jnp.full((padlen,), n, jnp.int32)])
    src2 = jnp.concatenate([src, src + npad])
    dst2d = dst.reshape(epad // CHUNK2, CHUNK2)

    xpad = jnp.zeros((npad, d), x.dtype).at[:n].set(x)
    aux = jnp.concatenate([jnp.zeros((CHUNK, DW), jnp.float32),
                           jnp.ones((CHUNK2, DW), jnp.float32)])

    deg_k = _make_deg_kernel(npad, epad)
    spass_k = _make_spass_kernel(npad, epad, dh)
    tc_y1 = _make_tc_y1(npad, d, br)
    tc_mid = _make_tc_mid(npad, d, br)
    tc_final = _make_tc_final(npad, d, br)

    dg = deg_k(aux, dst2d).reshape(NC, npad, DW)
    b1r = b1.reshape(1, d)
    b2r = b2.reshape(1, d)

    y1 = tc_y1(xpad, W1, dg)
    s1 = spass_k(y1.reshape(NC * npad, dh), src2, dst2d)
    y2 = tc_mid(s1.reshape(NC, npad, dh), dg, b1r, W2)
    s2 = spass_k(y2.reshape(NC * npad, dh), src2, dst2d)
    h = tc_final(s2.reshape(NC, npad, dh), dg, b2r)
    return h[:n]
